# baseline (speedup 1.0000x reference)
import jax
import jax.numpy as jnp
from jax.experimental import pallas as pl
from jax.experimental.pallas import tpu as pltpu

N = 100000
H = 128
HID = 2 * H
C = 128
G3 = 3 * HID

BLOCK = 1000


def _drnn_block_kernel(xa_ref, xf_ref, ph_ref, sh_ref, enc_ref,
                       wd_ih_ref, wd_hh_ref, ww_ih_ref, ww_hh_ref,
                       bd_ih_ref, bd_hh_ref, bw_ih_ref, bw_hh_ref,
                       w_h_ref, b_h_ref, w_pa_ref, w_pf_ref, b_p_ref,
                       h_out_ref, probs_out_ref):
    f32 = jnp.float32
    bf16 = jnp.bfloat16
    enc = enc_ref[...]

    def gru(x, h, wi_t, wh_t, bi, bh):
        gi = jnp.dot(x.astype(bf16), wi_t, preferred_element_type=f32) + bi
        gh = jnp.dot(h.astype(bf16), wh_t, preferred_element_type=f32) + bh
        r = jax.nn.sigmoid(gi[:, :HID] + gh[:, :HID])
        z = jax.nn.sigmoid(gi[:, HID:2 * HID] + gh[:, HID:2 * HID])
        n = jnp.tanh(gi[:, 2 * HID:] + r * gh[:, 2 * HID:])
        return (1.0 - z) * n + z * h

    ha = gru(xa_ref[...], jnp.concatenate([ph_ref[...], enc], axis=1),
             wd_ih_ref[...], wd_hh_ref[...], bd_ih_ref[...], bd_hh_ref[...])
    hf = gru(xf_ref[...], jnp.concatenate([sh_ref[...], enc], axis=1),
             ww_ih_ref[...], ww_hh_ref[...], bw_ih_ref[...], bw_hh_ref[...])

    hcat = jnp.concatenate([ha, hf], axis=1).astype(bf16)
    h_out_ref[...] = jnp.tanh(
        jnp.dot(hcat, w_h_ref[...], preferred_element_type=f32) + b_h_ref[...])

    pa = jnp.sum(ha * w_pa_ref[...], axis=1, keepdims=True)
    pf = jnp.sum(hf * w_pf_ref[...], axis=1, keepdims=True)
    probs_out_ref[...] = jax.nn.sigmoid(
        jnp.concatenate([pa, pf], axis=1) + b_p_ref[...])


def kernel(parent_output_label, sibling_output_label, parent_h, sibling_h, encoding,
           d_W_ih, d_W_hh, d_b_ih, d_b_hh,
           w_W_ih, w_W_hh, w_b_ih, w_b_hh,
           W_pa, b_pa, W_pf, b_pf, W_ha, b_ha, W_hf, b_hf):
    bf16 = jnp.bfloat16
    wd_ih_t = d_W_ih.T.astype(bf16)
    wd_hh_t = d_W_hh.T.astype(bf16)
    ww_ih_t = w_W_ih.T.astype(bf16)
    ww_hh_t = w_W_hh.T.astype(bf16)
    w_h = jnp.concatenate([W_ha.T, W_hf.T], axis=0).astype(bf16)
    b_h = (b_ha + b_hf).reshape(1, H)
    b_p = jnp.concatenate([b_pa, b_pf]).reshape(1, 2)

    row = lambda i: (i, 0)
    fixed = lambda i: (0, 0)
    act_spec = pl.BlockSpec((BLOCK, H), row)
    grid = N // BLOCK

    h_out, probs = pl.pallas_call(
        _drnn_block_kernel,
        grid=(grid,),
        in_specs=[
            act_spec, act_spec, act_spec, act_spec, act_spec,
            pl.BlockSpec((C, G3), fixed),
            pl.BlockSpec((HID, G3), fixed),
            pl.BlockSpec((C, G3), fixed),
            pl.BlockSpec((HID, G3), fixed),
            pl.BlockSpec((1, G3), fixed),
            pl.BlockSpec((1, G3), fixed),
            pl.BlockSpec((1, G3), fixed),
            pl.BlockSpec((1, G3), fixed),
            pl.BlockSpec((2 * HID, H), fixed),
            pl.BlockSpec((1, H), fixed),
            pl.BlockSpec((1, HID), fixed),
            pl.BlockSpec((1, HID), fixed),
            pl.BlockSpec((1, 2), fixed),
        ],
        out_specs=[
            pl.BlockSpec((BLOCK, H), row),
            pl.BlockSpec((BLOCK, 2), row),
        ],
        out_shape=[
            jax.ShapeDtypeStruct((N, H), jnp.float32),
            jax.ShapeDtypeStruct((N, 2), jnp.float32),
        ],
        compiler_params=pltpu.CompilerParams(
            dimension_semantics=("arbitrary",),
        ),
    )(parent_output_label, sibling_output_label, parent_h, sibling_h, encoding,
      wd_ih_t, wd_hh_t, ww_ih_t, ww_hh_t,
      d_b_ih.reshape(1, G3), d_b_hh.reshape(1, G3),
      w_b_ih.reshape(1, G3), w_b_hh.reshape(1, G3),
      w_h, b_h, W_pa, W_pf, b_p)
    return (h_out, probs)


# speedup vs baseline: 1.0471x; 1.0471x over previous
import jax
import jax.numpy as jnp
from jax.experimental import pallas as pl
from jax.experimental.pallas import tpu as pltpu

N = 100000
H = 128
HID = 2 * H
C = 128
G3 = 3 * HID

BLOCK = 2000


def _drnn_block_kernel(xa_ref, xf_ref, ph_ref, sh_ref, enc_ref,
                       wd_ih_ref, wd_hh_ref, ww_ih_ref, ww_hh_ref,
                       bd_ih_ref, bd_hh_ref, bw_ih_ref, bw_hh_ref,
                       w_h_ref, b_h_ref, w_pa_ref, w_pf_ref, b_p_ref,
                       h_out_ref, probs_out_ref):
    f32 = jnp.float32
    bf16 = jnp.bfloat16
    enc = enc_ref[...]

    def gru(x, h, wi_t, wh_t, bi, bh):
        gi = jnp.dot(x, wi_t, preferred_element_type=f32) + bi
        gh = jnp.dot(h, wh_t, preferred_element_type=f32) + bh
        r = jax.nn.sigmoid(gi[:, :HID] + gh[:, :HID])
        z = jax.nn.sigmoid(gi[:, HID:2 * HID] + gh[:, HID:2 * HID])
        n = jnp.tanh(gi[:, 2 * HID:] + r * gh[:, 2 * HID:])
        return (1.0 - z) * n + z * h

    ha = gru(xa_ref[...], jnp.concatenate([ph_ref[...], enc], axis=1),
             wd_ih_ref[...], wd_hh_ref[...], bd_ih_ref[...], bd_hh_ref[...])
    hf = gru(xf_ref[...], jnp.concatenate([sh_ref[...], enc], axis=1),
             ww_ih_ref[...], ww_hh_ref[...], bw_ih_ref[...], bw_hh_ref[...])

    hcat = jnp.concatenate([ha, hf], axis=1)
    h_out_ref[...] = jnp.tanh(
        jnp.dot(hcat, w_h_ref[...], preferred_element_type=f32) + b_h_ref[...])

    pa = jnp.sum(ha * w_pa_ref[...], axis=1, keepdims=True)
    pf = jnp.sum(hf * w_pf_ref[...], axis=1, keepdims=True)
    probs_out_ref[...] = jax.nn.sigmoid(
        jnp.concatenate([pa, pf], axis=1) + b_p_ref[...])


def kernel(parent_output_label, sibling_output_label, parent_h, sibling_h, encoding,
           d_W_ih, d_W_hh, d_b_ih, d_b_hh,
           w_W_ih, w_W_hh, w_b_ih, w_b_hh,
           W_pa, b_pa, W_pf, b_pf, W_ha, b_ha, W_hf, b_hf):
    wd_ih_t = d_W_ih.T
    wd_hh_t = d_W_hh.T
    ww_ih_t = w_W_ih.T
    ww_hh_t = w_W_hh.T
    w_h = jnp.concatenate([W_ha.T, W_hf.T], axis=0)
    b_h = (b_ha + b_hf).reshape(1, H)
    b_p = jnp.concatenate([b_pa, b_pf]).reshape(1, 2)

    row = lambda i: (i, 0)
    fixed = lambda i: (0, 0)
    act_spec = pl.BlockSpec((BLOCK, H), row)
    grid = N // BLOCK

    h_out, probs = pl.pallas_call(
        _drnn_block_kernel,
        grid=(grid,),
        in_specs=[
            act_spec, act_spec, act_spec, act_spec, act_spec,
            pl.BlockSpec((C, G3), fixed),
            pl.BlockSpec((HID, G3), fixed),
            pl.BlockSpec((C, G3), fixed),
            pl.BlockSpec((HID, G3), fixed),
            pl.BlockSpec((1, G3), fixed),
            pl.BlockSpec((1, G3), fixed),
            pl.BlockSpec((1, G3), fixed),
            pl.BlockSpec((1, G3), fixed),
            pl.BlockSpec((2 * HID, H), fixed),
            pl.BlockSpec((1, H), fixed),
            pl.BlockSpec((1, HID), fixed),
            pl.BlockSpec((1, HID), fixed),
            pl.BlockSpec((1, 2), fixed),
        ],
        out_specs=[
            pl.BlockSpec((BLOCK, H), row),
            pl.BlockSpec((BLOCK, 2), row),
        ],
        out_shape=[
            jax.ShapeDtypeStruct((N, H), jnp.float32),
            jax.ShapeDtypeStruct((N, 2), jnp.float32),
        ],
        compiler_params=pltpu.CompilerParams(
            dimension_semantics=("arbitrary",),
        ),
    )(parent_output_label, sibling_output_label, parent_h, sibling_h, encoding,
      wd_ih_t, wd_hh_t, ww_ih_t, ww_hh_t,
      d_b_ih.reshape(1, G3), d_b_hh.reshape(1, G3),
      w_b_ih.reshape(1, G3), w_b_hh.reshape(1, G3),
      w_h, b_h, W_pa, W_pf, b_p)
    return (h_out, probs)
